# baseline (speedup 1.0000x reference)
import functools

import jax
import jax.numpy as jnp
import numpy as np
from jax import lax
from jax.experimental import pallas as pl
from jax.experimental.pallas import tpu as pltpu
from jax.experimental.pallas import tpu_sc as plsc

_N = 10000
_E = 160000
_D = 128
_DA = 16
_DEA = 4
_DEE = 16
_DQK = 32
_H = 8

_NC = 2
_NS = 16
_NW = _NC * _NS
_EP = 163840
_EPW = _EP // _NW
_CH = 128
_NCH = _EPW // _CH
_NP = 10240
_STRIPE = _NP // _NS
_HF = _EP // 2
_NCHH = _HF // _CH
_CPW = _NCHH // _NW

_LOG2 = float(np.log(2.0))


def _cutoff_log_half() -> float:
    el = np.sqrt(np.float32(1e-12))
    xc = np.float32(10.0) * (np.float32(1.0) - el / np.float32(5.0))
    return float(-0.5 / float(xc))


def _pre_body(nf_ref, na_ref, wq_ref, wsc_ref, q_ref, sc_ref):
    nf = nf_ref[...]
    na = na_ref[...]
    q_ref[...] = jnp.dot(nf, wq_ref[...], preferred_element_type=jnp.float32)
    acc = jnp.zeros((nf.shape[0], _D), jnp.float32)
    for b in range(_DA):
        acc = acc + jnp.dot(na[:, b:b + 1] * nf, wsc_ref[:, b, :],
                            preferred_element_type=jnp.float32)
    sc_ref[...] = acc


def _tc_pre(nf, na, wq, wsc):
    bn = 1000
    grid = _N // bn
    return pl.pallas_call(
        _pre_body,
        grid=(grid,),
        in_specs=[
            pl.BlockSpec((bn, _D), lambda i: (i, 0)),
            pl.BlockSpec((bn, _DA), lambda i: (i, 0)),
            pl.BlockSpec((_D, _D), lambda i: (0, 0)),
            pl.BlockSpec((_D, _DA, _D), lambda i: (0, 0, 0)),
        ],
        out_specs=[
            pl.BlockSpec((bn, _D), lambda i: (i, 0)),
            pl.BlockSpec((bn, _D), lambda i: (i, 0)),
        ],
        out_shape=[
            jax.ShapeDtypeStruct((_N, _D), jnp.float32),
            jax.ShapeDtypeStruct((_N, _D), jnp.float32),
        ],
    )(nf, na, wq, wsc)


def _sc_gather(nf, q, src1, dst1, half):
    mesh = plsc.VectorSubcoreMesh(core_axis_name="c", subcore_axis_name="s")

    @functools.partial(
        pl.kernel,
        mesh=mesh,
        out_type=[
            jax.ShapeDtypeStruct((_HF, _D), jnp.float32),
            jax.ShapeDtypeStruct((_HF, _D), jnp.float32),
        ],
        scratch_types=[
            pltpu.VMEM((_CPW * _CH,), jnp.int32),
            pltpu.VMEM((_CPW * _CH,), jnp.int32),
            pltpu.VMEM((_CH, _D), jnp.float32),
            pltpu.VMEM((_CH, _D), jnp.float32),
            pltpu.VMEM((_CH, _D), jnp.float32),
            pltpu.VMEM((_CH, _D), jnp.float32),
            pltpu.VMEM((_CH, _D), jnp.float32),
            pltpu.VMEM((_CH, _D), jnp.float32),
            pltpu.SemaphoreType.DMA,
            pltpu.SemaphoreType.DMA,
            pltpu.SemaphoreType.DMA,
            pltpu.SemaphoreType.DMA,
            pltpu.SemaphoreType.DMA,
            pltpu.SemaphoreType.DMA,
        ],
    )
    def gather(nf_hbm, q_hbm, src_hbm, dst_hbm, xs_out, qd_out,
               idx_s, idx_d, rx0, rx1, rx2, rq0, rq1, rq2,
               gs0, gs1, gs2, ws0, ws1, ws2):
        wid = lax.axis_index("c") * _NS + lax.axis_index("s")
        rx = (rx0, rx1, rx2)
        rq = (rq0, rq1, rq2)
        gs = (gs0, gs1, gs2)
        ws = (ws0, ws1, ws2)
        lbase = wid * _CPW * _CH
        gbase = half * _HF + lbase
        nch = _CPW

        pltpu.sync_copy(src_hbm.at[pl.ds(gbase, nch * _CH)], idx_s)
        pltpu.sync_copy(dst_hbm.at[pl.ds(gbase, nch * _CH)], idx_d)

        def fire(i, b):
            isl = idx_s.at[pl.ds(i * _CH, _CH)]
            idl = idx_d.at[pl.ds(i * _CH, _CH)]
            pltpu.async_copy(nf_hbm.at[isl], rx[b], gs[b])
            pltpu.async_copy(q_hbm.at[idl], rq[b], gs[b])

        def handle(i, b):
            off = pl.multiple_of(lbase + i * _CH, 128)
            pltpu.make_async_copy(nf_hbm.at[pl.ds(0, _CH)],
                                  rx[b], gs[b]).wait()
            pltpu.make_async_copy(q_hbm.at[pl.ds(0, _CH)],
                                  rq[b], gs[b]).wait()
            pltpu.async_copy(rx[b], xs_out.at[pl.ds(off, _CH)], ws[b])
            pltpu.async_copy(rq[b], qd_out.at[pl.ds(off, _CH)], ws[b])
            pltpu.make_async_copy(rx[b], xs_out.at[pl.ds(off, _CH)],
                                  ws[b]).wait()
            pltpu.make_async_copy(rq[b], qd_out.at[pl.ds(off, _CH)],
                                  ws[b]).wait()

        fire(0, 0)
        fire(1, 1)
        fire(2, 2)

        def body(g, _):
            for b in (0, 1, 2):
                i = 3 * g + b
                handle(i, b)

                @pl.when(i + 3 < nch)
                def _next():
                    fire(i + 3, b)
            return _

        lax.fori_loop(0, nch // 3, body, None)
        for t in range(nch % 3):
            handle(3 * (nch // 3) + t, t)

    return gather(nf, q, src1, dst1)


def _edge_body(ee_ref, ea_ref, xs_ref, pd_ref, w1_ref, rep_ref, til_ref,
               ak_ref, av_ref, sv_ref, s2_ref, *, logc_half, ebase):
    ee = ee_ref[...]
    ea = ea_ref[...]
    h = jnp.dot(ee, w1_ref[...], preferred_element_type=jnp.float32)
    h = jnp.logaddexp(h, 0.0) - _LOG2
    hexp = jnp.dot(h, rep_ref[...], preferred_element_type=jnp.float32)
    eat = jnp.dot(ea, til_ref[...], preferred_element_type=jnp.float32)
    g = hexp * eat
    tk = jnp.dot(g, ak_ref[...], preferred_element_type=jnp.float32)
    tv = jnp.dot(g, av_ref[...], preferred_element_type=jnp.float32)
    xs = xs_ref[...]
    dot = jnp.sum(xs * tk * pd_ref[...], axis=1, keepdims=True)
    mid = xs * tv
    b = dot.shape[0]
    eidx = (ebase + pl.program_id(0) * b
            + jax.lax.broadcasted_iota(jnp.int32, (b, 1), 0))
    live = eidx < _E
    s = jnp.where(live, jnp.exp(0.5 * dot + logc_half), 0.0)
    sv_ref[...] = s * mid
    s2_ref[...] = (s * s).T


def _tc_edge(ee, ea, xs, pd, w1, rep, til, ak, av, half):
    be = 2048
    grid = _HF // be
    body = functools.partial(_edge_body, logc_half=_cutoff_log_half(),
                             ebase=half * _HF)
    return pl.pallas_call(
        body,
        grid=(grid,),
        in_specs=[
            pl.BlockSpec((be, _DEE), lambda i: (i, 0)),
            pl.BlockSpec((be, _DEA), lambda i: (i, 0)),
            pl.BlockSpec((be, _D), lambda i: (i, 0)),
            pl.BlockSpec((be, _D), lambda i: (i, 0)),
            pl.BlockSpec((_DEE, 2 * _H), lambda i: (0, 0)),
            pl.BlockSpec((2 * _H, 64), lambda i: (0, 0)),
            pl.BlockSpec((_DEA, 64), lambda i: (0, 0)),
            pl.BlockSpec((64, _D), lambda i: (0, 0)),
            pl.BlockSpec((64, _D), lambda i: (0, 0)),
        ],
        out_specs=[
            pl.BlockSpec((be, _D), lambda i: (i, 0)),
            pl.BlockSpec((1, be), lambda i: (0, i)),
        ],
        out_shape=[
            jax.ShapeDtypeStruct((_HF, _D), jnp.float32),
            jax.ShapeDtypeStruct((1, _HF), jnp.float32),
        ],
    )(ee, ea, xs, pd, w1, rep, til, ak, av)


def _sc_scatter(sv0, s20, sv1, s21, dst, zrow, zrow1):
    mesh = plsc.VectorSubcoreMesh(core_axis_name="c", subcore_axis_name="s")

    @functools.partial(
        pl.kernel,
        mesh=mesh,
        out_type=[
            jax.ShapeDtypeStruct((_NC, _NP, _D), jnp.float32),
            jax.ShapeDtypeStruct((_NC, _NP), jnp.float32),
        ],
        scratch_types=[
            pltpu.VMEM((_NCH, _CH), jnp.int32),
            pltpu.VMEM((_CH, _D), jnp.float32),
            pltpu.VMEM((_CH, _D), jnp.float32),
            pltpu.VMEM((_CH,), jnp.float32),
            pltpu.VMEM((_CH,), jnp.float32),
            pltpu.VMEM_SHARED((_NP, _D), jnp.float32),
            pltpu.VMEM_SHARED((_NP,), jnp.float32),
            pltpu.SemaphoreType.DMA,
            pltpu.SemaphoreType.DMA,
            pltpu.SemaphoreType.DMA,
            pltpu.SemaphoreType.DMA,
        ],
    )
    def scatter(sv0_hbm, s20_hbm, sv1_hbm, s21_hbm, dst_hbm,
                zr_hbm, zr1_hbm, num_out, z_out,
                idx_v, rv0, rv1, s2a, s2c, num_sh, z_sh,
                ls0, ls1, ss0, ss1):
        cid = lax.axis_index("c")
        sid = lax.axis_index("s")
        wid = cid * _NS + sid
        lbase = sid * _EPW
        cbase = wid * _NCH
        rbase = pl.multiple_of(sid * _STRIPE, 8)
        rv = (rv0, rv1)
        s2b = (s2a, s2c)
        ls = (ls0, ls1)
        ss = (ss0, ss1)

        pltpu.sync_copy(zr_hbm, num_sh.at[pl.ds(rbase, _STRIPE)])
        pltpu.sync_copy(zr1_hbm.at[pl.ds(rbase, _STRIPE)],
                        z_sh.at[pl.ds(rbase, _STRIPE)])
        pltpu.sync_copy(dst_hbm.at[pl.ds(cbase, _NCH)], idx_v)
        plsc.subcore_barrier()

        def run(sv_hbm, s2_hbm):
            def fire_load(i, b):
                off = pl.multiple_of(lbase + i * _CH, 128)
                pltpu.async_copy(sv_hbm.at[pl.ds(off, _CH)], rv[b], ls[b])
                pltpu.async_copy(s2_hbm.at[0, pl.ds(off, _CH)],
                                 s2b[b], ls[b])

            fire_load(0, 0)
            fire_load(1, 1)

            def body(g, _):
                for b in (0, 1):
                    i = 2 * g + b
                    off = pl.multiple_of(lbase + i * _CH, 128)
                    pltpu.make_async_copy(sv_hbm.at[pl.ds(off, _CH)],
                                          rv[b], ls[b]).wait()
                    pltpu.make_async_copy(s2_hbm.at[0, pl.ds(off, _CH)],
                                          s2b[b], ls[b]).wait()
                    pltpu.async_copy(rv[b], num_sh.at[idx_v.at[i]], ss[b],
                                     add=True)
                    pltpu.async_copy(s2b[b], z_sh.at[idx_v.at[i]], ss[b],
                                     add=True)
                    pltpu.make_async_copy(rv[b], num_sh.at[idx_v.at[i]],
                                          ss[b]).wait()
                    pltpu.make_async_copy(s2b[b], z_sh.at[idx_v.at[i]],
                                          ss[b]).wait()

                    @pl.when(i + 2 < _NCH)
                    def _next():
                        fire_load(i + 2, b)
                return _

            lax.fori_loop(0, _NCH // 2, body, None)

        @pl.when(cid == 0)
        def _h0():
            run(sv0_hbm, s20_hbm)

        @pl.when(cid == 1)
        def _h1():
            run(sv1_hbm, s21_hbm)

        plsc.subcore_barrier()

        pltpu.sync_copy(num_sh.at[pl.ds(rbase, _STRIPE)],
                        num_out.at[cid, pl.ds(rbase, _STRIPE)])
        pltpu.sync_copy(z_sh.at[pl.ds(rbase, _STRIPE)],
                        z_out.at[cid, pl.ds(rbase, _STRIPE)])

    return scatter(sv0, s20, sv1, s21, dst, zrow, zrow1)


def _post_body(num_ref, z0_ref, z1_ref, sc_ref, wv_ref, out_ref):
    num = num_ref[0] + num_ref[1]
    nv = jnp.dot(num, wv_ref[...], preferred_element_type=jnp.float32)
    z = z0_ref[...] + z1_ref[...]
    z = jnp.where(z == 0.0, 1.0, z)
    out_ref[...] = nv / jnp.sqrt(z) + sc_ref[...]


def _tc_post(num, z0, z1, sc, wv):
    bn = 1000
    grid = _N // bn
    return pl.pallas_call(
        _post_body,
        grid=(grid,),
        in_specs=[
            pl.BlockSpec((_NC, bn, _D), lambda i: (0, i, 0)),
            pl.BlockSpec((bn, 1), lambda i: (i, 0)),
            pl.BlockSpec((bn, 1), lambda i: (i, 0)),
            pl.BlockSpec((bn, _D), lambda i: (i, 0)),
            pl.BlockSpec((_D, _D), lambda i: (0, 0)),
        ],
        out_specs=pl.BlockSpec((bn, _D), lambda i: (i, 0)),
        out_shape=jax.ShapeDtypeStruct((_N, _D), jnp.float32),
    )(num, z0, z1, sc, wv)


def kernel(node_features, node_attrs, edge_embedding, edge_attrs, positions,
           edge_index, Wq, W1k, W2k, W1v, W2v, Wk, Wv, Wdot, Wsc):
    del positions
    f32 = jnp.float32
    src = edge_index[0].astype(jnp.int32)
    dst = edge_index[1].astype(jnp.int32)

    w1 = jnp.concatenate([W1k, W1v], axis=1).astype(f32)
    rep = np.zeros((2 * _H, 64), np.float32)
    til = np.zeros((_DEA, 64), np.float32)
    for hh in range(_H):
        for vv in range(_DEA):
            rep[hh, hh * _DEA + vv] = 1.0
            rep[_H + hh, 32 + hh * _DEA + vv] = 1.0
            til[vv, hh * _DEA + vv] = 1.0
            til[vv, 32 + hh * _DEA + vv] = 1.0
    rep = jnp.asarray(rep)
    til = jnp.asarray(til)
    akm = W2k.reshape(_H, _D, _DEA).transpose(0, 2, 1).reshape(32, _D)
    avm = W2v.reshape(_H, _D, _DEA).transpose(0, 2, 1).reshape(32, _D)
    scale = 1.0 / np.sqrt(_DEA)
    ak = jnp.zeros((64, _D), f32).at[:32].set(akm * scale)
    av = jnp.zeros((64, _D), f32).at[32:].set(avm * scale)

    zrow = jnp.zeros((_STRIPE, _D), f32)
    zrow1 = jnp.zeros((_NP,), f32)

    padn = _EP - _E
    src1 = jnp.pad(src, (0, padn))
    dst1 = jnp.pad(dst, (0, padn))
    dst2d = dst1.reshape(_EP // _CH, _CH)
    eep = jnp.pad(edge_embedding, ((0, padn), (0, 0)))
    eap = jnp.pad(edge_attrs, ((0, padn), (0, 0)))

    wq128 = (Wq @ Wdot @ Wk.T).astype(f32)
    p, sc = _tc_pre(node_features, node_attrs, wq128, Wsc)
    xs0, pd0 = _sc_gather(node_features, p, src1, dst1, 0)
    xs1, pd1 = _sc_gather(node_features, p, src1, dst1, 1)
    sv0, s20 = _tc_edge(eep[:_HF], eap[:_HF], xs0, pd0,
                        w1, rep, til, ak, av, 0)
    sv1, s21 = _tc_edge(eep[_HF:], eap[_HF:], xs1, pd1,
                        w1, rep, til, ak, av, 1)
    num, z = _sc_scatter(sv0, s20, sv1, s21, dst2d, zrow, zrow1)
    z0 = z[0, :_N, None]
    z1 = z[1, :_N, None]
    return _tc_post(num, z0, z1, sc, Wv)


# speedup vs baseline: 1.4729x; 1.4729x over previous
import functools

import jax
import jax.numpy as jnp
import numpy as np
from jax import lax
from jax.experimental import pallas as pl
from jax.experimental.pallas import tpu as pltpu
from jax.experimental.pallas import tpu_sc as plsc

_N = 10000
_E = 160000
_D = 128
_DA = 16
_DEA = 4
_DEE = 16
_DQK = 32
_H = 8

_NC = 2
_NS = 16
_NW = _NC * _NS
_EP = 163840
_EPW = _EP // _NW
_CH = 128
_NCH = _EPW // _CH
_NP = 10240
_STRIPE = _NP // _NS
_HF = _EP // 2
_NCHH = _HF // _CH
_CPW = _NCHH // _NW

_LOG2 = float(np.log(2.0))


def _cutoff_log_half() -> float:
    el = np.sqrt(np.float32(1e-12))
    xc = np.float32(10.0) * (np.float32(1.0) - el / np.float32(5.0))
    return float(-0.5 / float(xc))


def _pre_body(nf_ref, na_ref, wq_ref, wsc_ref, q_ref, sc_ref):
    nf = nf_ref[...]
    na = na_ref[...]
    q_ref[...] = jnp.dot(nf, wq_ref[...], preferred_element_type=jnp.float32)
    acc = jnp.zeros((nf.shape[0], _D), jnp.float32)
    for b in range(_DA):
        acc = acc + jnp.dot(na[:, b:b + 1] * nf, wsc_ref[:, b, :],
                            preferred_element_type=jnp.float32)
    sc_ref[...] = acc


def _tc_pre(nf, na, wq, wsc):
    bn = 1000
    grid = _N // bn
    return pl.pallas_call(
        _pre_body,
        grid=(grid,),
        in_specs=[
            pl.BlockSpec((bn, _D), lambda i: (i, 0)),
            pl.BlockSpec((bn, _DA), lambda i: (i, 0)),
            pl.BlockSpec((_D, _D), lambda i: (0, 0)),
            pl.BlockSpec((_D, _DA, _D), lambda i: (0, 0, 0)),
        ],
        out_specs=[
            pl.BlockSpec((bn, _D), lambda i: (i, 0)),
            pl.BlockSpec((bn, _D), lambda i: (i, 0)),
        ],
        out_shape=[
            jax.ShapeDtypeStruct((_N, _D), jnp.float32),
            jax.ShapeDtypeStruct((_N, _D), jnp.float32),
        ],
    )(nf, na, wq, wsc)


def _sc_gather(nf, q, src1, dst1, half):
    mesh = plsc.VectorSubcoreMesh(core_axis_name="c", subcore_axis_name="s")

    nchh = _NCHH // _NS

    @functools.partial(
        pl.kernel,
        mesh=mesh,
        out_type=[
            jax.ShapeDtypeStruct((_HF, _D), jnp.float32),
            jax.ShapeDtypeStruct((_HF, _D), jnp.float32),
        ],
        scratch_types=[
            pltpu.VMEM((nchh * _CH,), jnp.int32),
            pltpu.VMEM((_CH, _D), jnp.float32),
            pltpu.VMEM((_CH, _D), jnp.float32),
            pltpu.VMEM_SHARED((_N, _D), jnp.float32),
            pltpu.SemaphoreType.DMA,
            pltpu.SemaphoreType.DMA,
            pltpu.SemaphoreType.DMA,
            pltpu.SemaphoreType.DMA,
        ],
    )
    def gather(nf_hbm, q_hbm, src_hbm, dst_hbm, xs_out, qd_out,
               idx_v, r0, r1, tab_sh,
               gs0, gs1, ws0, ws1):
        cid = lax.axis_index("c")
        sid = lax.axis_index("s")
        rx = (r0, r1)
        gs = (gs0, gs1)
        ws = (ws0, ws1)
        tb = pl.multiple_of(sid * 640, 8)

        def stage(tab_hbm):
            @pl.when(sid < _NS - 1)
            def _full():
                pltpu.sync_copy(tab_hbm.at[pl.ds(tb, 640)],
                                tab_sh.at[pl.ds(tb, 640)])

            @pl.when(sid == _NS - 1)
            def _last():
                pltpu.sync_copy(tab_hbm.at[pl.ds(tb, _N - 15 * 640)],
                                tab_sh.at[pl.ds(tb, _N - 15 * 640)])

        @pl.when(cid == 0)
        def _s0():
            stage(nf_hbm)

        @pl.when(cid == 1)
        def _s1():
            stage(q_hbm)

        plsc.subcore_barrier()

        def run(ih_hbm, out_hbm):
            lbase = sid * nchh * _CH
            gbase = half * _HF + lbase
            pltpu.sync_copy(ih_hbm.at[pl.ds(gbase, nchh * _CH)], idx_v)

            def fire(i, b):
                isl = idx_v.at[pl.ds(i * _CH, _CH)]
                pltpu.async_copy(tab_sh.at[isl], rx[b], gs[b])

            def handle(i, b):
                off = pl.multiple_of(lbase + i * _CH, 128)
                pltpu.make_async_copy(nf_hbm.at[pl.ds(0, _CH)],
                                      rx[b], gs[b]).wait()
                pltpu.async_copy(rx[b], out_hbm.at[pl.ds(off, _CH)], ws[b])
                pltpu.make_async_copy(rx[b], out_hbm.at[pl.ds(off, _CH)],
                                      ws[b]).wait()

            fire(0, 0)
            fire(1, 1)

            def body(g, _):
                for b in (0, 1):
                    i = 2 * g + b
                    handle(i, b)

                    @pl.when(i + 2 < nchh)
                    def _next():
                        fire(i + 2, b)
                return _

            lax.fori_loop(0, nchh // 2, body, None)

        @pl.when(cid == 0)
        def _g0():
            run(src_hbm, xs_out)

        @pl.when(cid == 1)
        def _g1():
            run(dst_hbm, qd_out)

    return gather(nf, q, src1, dst1)


def _edge_body(ee_ref, ea_ref, xs_ref, pd_ref, w1_ref, rep_ref, til_ref,
               ak_ref, av_ref, sv_ref, s2_ref, *, logc_half, ebase):
    ee = ee_ref[...]
    ea = ea_ref[...]
    h = jnp.dot(ee, w1_ref[...], preferred_element_type=jnp.float32)
    h = jnp.logaddexp(h, 0.0) - _LOG2
    hexp = jnp.dot(h, rep_ref[...], preferred_element_type=jnp.float32)
    eat = jnp.dot(ea, til_ref[...], preferred_element_type=jnp.float32)
    g = hexp * eat
    tk = jnp.dot(g, ak_ref[...], preferred_element_type=jnp.float32)
    tv = jnp.dot(g, av_ref[...], preferred_element_type=jnp.float32)
    xs = xs_ref[...]
    dot = jnp.sum(xs * tk * pd_ref[...], axis=1, keepdims=True)
    mid = xs * tv
    b = dot.shape[0]
    eidx = (ebase + pl.program_id(0) * b
            + jax.lax.broadcasted_iota(jnp.int32, (b, 1), 0))
    live = eidx < _E
    s = jnp.where(live, jnp.exp(0.5 * dot + logc_half), 0.0)
    sv_ref[...] = s * mid
    s2_ref[...] = (s * s).T


def _tc_edge(ee, ea, xs, pd, w1, rep, til, ak, av, half):
    be = 2048
    grid = _HF // be
    body = functools.partial(_edge_body, logc_half=_cutoff_log_half(),
                             ebase=half * _HF)
    hb = half * (_HF // be)
    return pl.pallas_call(
        body,
        grid=(grid,),
        in_specs=[
            pl.BlockSpec((be, _DEE), lambda i: (i + hb, 0)),
            pl.BlockSpec((be, _DEA), lambda i: (i + hb, 0)),
            pl.BlockSpec((be, _D), lambda i: (i, 0)),
            pl.BlockSpec((be, _D), lambda i: (i, 0)),
            pl.BlockSpec((_DEE, 2 * _H), lambda i: (0, 0)),
            pl.BlockSpec((2 * _H, 64), lambda i: (0, 0)),
            pl.BlockSpec((_DEA, 64), lambda i: (0, 0)),
            pl.BlockSpec((64, _D), lambda i: (0, 0)),
            pl.BlockSpec((64, _D), lambda i: (0, 0)),
        ],
        out_specs=[
            pl.BlockSpec((be, _D), lambda i: (i, 0)),
            pl.BlockSpec((1, be), lambda i: (0, i)),
        ],
        out_shape=[
            jax.ShapeDtypeStruct((_HF, _D), jnp.float32),
            jax.ShapeDtypeStruct((1, _HF), jnp.float32),
        ],
    )(ee, ea, xs, pd, w1, rep, til, ak, av)


def _sc_scatter(sv0, s20, sv1, s21, dst, zrow, zrow1):
    mesh = plsc.VectorSubcoreMesh(core_axis_name="c", subcore_axis_name="s")

    @functools.partial(
        pl.kernel,
        mesh=mesh,
        out_type=[
            jax.ShapeDtypeStruct((_NC, _NP, _D), jnp.float32),
            jax.ShapeDtypeStruct((_NC, _NP), jnp.float32),
        ],
        scratch_types=[
            pltpu.VMEM((_NCH, _CH), jnp.int32),
            pltpu.VMEM((_CH, _D), jnp.float32),
            pltpu.VMEM((_CH, _D), jnp.float32),
            pltpu.VMEM((_CH,), jnp.float32),
            pltpu.VMEM((_CH,), jnp.float32),
            pltpu.VMEM_SHARED((_NP, _D), jnp.float32),
            pltpu.VMEM_SHARED((_NP,), jnp.float32),
            pltpu.SemaphoreType.DMA,
            pltpu.SemaphoreType.DMA,
            pltpu.SemaphoreType.DMA,
            pltpu.SemaphoreType.DMA,
        ],
    )
    def scatter(sv0_hbm, s20_hbm, sv1_hbm, s21_hbm, dst_hbm,
                zr_hbm, zr1_hbm, num_out, z_out,
                idx_v, rv0, rv1, s2a, s2c, num_sh, z_sh,
                ls0, ls1, ss0, ss1):
        cid = lax.axis_index("c")
        sid = lax.axis_index("s")
        wid = cid * _NS + sid
        lbase = sid * _EPW
        cbase = wid * _NCH
        rbase = pl.multiple_of(sid * _STRIPE, 8)
        rv = (rv0, rv1)
        s2b = (s2a, s2c)
        ls = (ls0, ls1)
        ss = (ss0, ss1)

        pltpu.sync_copy(zr_hbm, num_sh.at[pl.ds(rbase, _STRIPE)])
        pltpu.sync_copy(zr1_hbm.at[pl.ds(rbase, _STRIPE)],
                        z_sh.at[pl.ds(rbase, _STRIPE)])
        pltpu.sync_copy(dst_hbm.at[pl.ds(cbase, _NCH)], idx_v)
        plsc.subcore_barrier()

        def run(sv_hbm, s2_hbm):
            def fire_load(i, b):
                off = pl.multiple_of(lbase + i * _CH, 128)
                pltpu.async_copy(sv_hbm.at[pl.ds(off, _CH)], rv[b], ls[b])
                pltpu.async_copy(s2_hbm.at[0, pl.ds(off, _CH)],
                                 s2b[b], ls[b])

            fire_load(0, 0)
            fire_load(1, 1)

            def body(g, _):
                for b in (0, 1):
                    i = 2 * g + b
                    off = pl.multiple_of(lbase + i * _CH, 128)
                    pltpu.make_async_copy(sv_hbm.at[pl.ds(off, _CH)],
                                          rv[b], ls[b]).wait()
                    pltpu.make_async_copy(s2_hbm.at[0, pl.ds(off, _CH)],
                                          s2b[b], ls[b]).wait()
                    pltpu.async_copy(rv[b], num_sh.at[idx_v.at[i]], ss[b],
                                     add=True)
                    pltpu.async_copy(s2b[b], z_sh.at[idx_v.at[i]], ss[b],
                                     add=True)
                    pltpu.make_async_copy(rv[b], num_sh.at[idx_v.at[i]],
                                          ss[b]).wait()
                    pltpu.make_async_copy(s2b[b], z_sh.at[idx_v.at[i]],
                                          ss[b]).wait()

                    @pl.when(i + 2 < _NCH)
                    def _next():
                        fire_load(i + 2, b)
                return _

            lax.fori_loop(0, _NCH // 2, body, None)

        @pl.when(cid == 0)
        def _h0():
            run(sv0_hbm, s20_hbm)

        @pl.when(cid == 1)
        def _h1():
            run(sv1_hbm, s21_hbm)

        plsc.subcore_barrier()

        pltpu.sync_copy(num_sh.at[pl.ds(rbase, _STRIPE)],
                        num_out.at[cid, pl.ds(rbase, _STRIPE)])
        pltpu.sync_copy(z_sh.at[pl.ds(rbase, _STRIPE)],
                        z_out.at[cid, pl.ds(rbase, _STRIPE)])

    return scatter(sv0, s20, sv1, s21, dst, zrow, zrow1)


def _post_body(num_ref, z0_ref, z1_ref, sc_ref, wv_ref, out_ref):
    num = num_ref[0] + num_ref[1]
    nv = jnp.dot(num, wv_ref[...], preferred_element_type=jnp.float32)
    z = z0_ref[...] + z1_ref[...]
    z = jnp.where(z == 0.0, 1.0, z)
    out_ref[...] = nv / jnp.sqrt(z) + sc_ref[...]


def _tc_post(num, z0, z1, sc, wv):
    bn = 1000
    grid = _N // bn
    return pl.pallas_call(
        _post_body,
        grid=(grid,),
        in_specs=[
            pl.BlockSpec((_NC, bn, _D), lambda i: (0, i, 0)),
            pl.BlockSpec((bn, 1), lambda i: (i, 0)),
            pl.BlockSpec((bn, 1), lambda i: (i, 0)),
            pl.BlockSpec((bn, _D), lambda i: (i, 0)),
            pl.BlockSpec((_D, _D), lambda i: (0, 0)),
        ],
        out_specs=pl.BlockSpec((bn, _D), lambda i: (i, 0)),
        out_shape=jax.ShapeDtypeStruct((_N, _D), jnp.float32),
    )(num, z0, z1, sc, wv)


def kernel(node_features, node_attrs, edge_embedding, edge_attrs, positions,
           edge_index, Wq, W1k, W2k, W1v, W2v, Wk, Wv, Wdot, Wsc):
    del positions
    f32 = jnp.float32
    src = edge_index[0].astype(jnp.int32)
    dst = edge_index[1].astype(jnp.int32)

    w1 = jnp.concatenate([W1k, W1v], axis=1).astype(f32)
    rep = np.zeros((2 * _H, 64), np.float32)
    til = np.zeros((_DEA, 64), np.float32)
    for hh in range(_H):
        for vv in range(_DEA):
            rep[hh, hh * _DEA + vv] = 1.0
            rep[_H + hh, 32 + hh * _DEA + vv] = 1.0
            til[vv, hh * _DEA + vv] = 1.0
            til[vv, 32 + hh * _DEA + vv] = 1.0
    rep = jnp.asarray(rep)
    til = jnp.asarray(til)
    akm = W2k.reshape(_H, _D, _DEA).transpose(0, 2, 1).reshape(32, _D)
    avm = W2v.reshape(_H, _D, _DEA).transpose(0, 2, 1).reshape(32, _D)
    scale = 1.0 / np.sqrt(_DEA)
    ak = jnp.zeros((64, _D), f32).at[:32].set(akm * scale)
    av = jnp.zeros((64, _D), f32).at[32:].set(avm * scale)

    zrow = jnp.zeros((_STRIPE, _D), f32)
    zrow1 = jnp.zeros((_NP,), f32)

    padn = _EP - _E
    src1 = jnp.pad(src, (0, padn))
    dst1 = jnp.pad(dst, (0, padn))
    dst2d = dst1.reshape(_EP // _CH, _CH)
    eep = jnp.pad(edge_embedding, ((0, padn), (0, 0)))
    eap = jnp.pad(edge_attrs, ((0, padn), (0, 0)))

    wq128 = (Wq @ Wdot @ Wk.T).astype(f32)
    p, sc = _tc_pre(node_features, node_attrs, wq128, Wsc)
    xs0, pd0 = _sc_gather(node_features, p, src1, dst1, 0)
    xs1, pd1 = _sc_gather(node_features, p, src1, dst1, 1)
    sv0, s20 = _tc_edge(eep, eap, xs0, pd0, w1, rep, til, ak, av, 0)
    sv1, s21 = _tc_edge(eep, eap, xs1, pd1, w1, rep, til, ak, av, 1)
    num, z = _sc_scatter(sv0, s20, sv1, s21, dst2d, zrow, zrow1)
    z0 = z[0, :_N, None]
    z1 = z[1, :_N, None]
    return _tc_post(num, z0, z1, sc, Wv)


# speedup vs baseline: 1.5507x; 1.0528x over previous
import functools

import jax
import jax.numpy as jnp
import numpy as np
from jax import lax
from jax.experimental import pallas as pl
from jax.experimental.pallas import tpu as pltpu
from jax.experimental.pallas import tpu_sc as plsc

_N = 10000
_E = 160000
_D = 128
_DA = 16
_DEA = 4
_DEE = 16
_DQK = 32
_H = 8

_NC = 2
_NS = 16
_NW = _NC * _NS
_EP = 163840
_EPW = _EP // _NW
_CH = 128
_NCH = _EPW // _CH
_NP = 10240
_STRIPE = _NP // _NS
_HF = _EP // 2
_NCHH = _HF // _CH
_CPW = _NCHH // _NW

_LOG2 = float(np.log(2.0))


def _cutoff_log_half() -> float:
    el = np.sqrt(np.float32(1e-12))
    xc = np.float32(10.0) * (np.float32(1.0) - el / np.float32(5.0))
    return float(-0.5 / float(xc))


def _ptab_body(nf_ref, wq_ref, q_ref):
    q_ref[...] = jnp.dot(nf_ref[...], wq_ref[...],
                         preferred_element_type=jnp.float32)


def _tc_ptab(nf, wq):
    bn = 1000
    grid = _N // bn
    return pl.pallas_call(
        _ptab_body,
        grid=(grid,),
        in_specs=[
            pl.BlockSpec((bn, _D), lambda i: (i, 0)),
            pl.BlockSpec((_D, _D), lambda i: (0, 0)),
        ],
        out_specs=pl.BlockSpec((bn, _D), lambda i: (i, 0)),
        out_shape=jax.ShapeDtypeStruct((_N, _D), jnp.float32),
    )(nf, wq)


def _sconn_body(nf_ref, na_ref, wsc_ref, sc_ref):
    nf = nf_ref[...]
    na = na_ref[...]
    acc = jnp.zeros((nf.shape[0], _D), jnp.float32)
    for b in range(_DA):
        acc = acc + jnp.dot(na[:, b:b + 1] * nf, wsc_ref[:, b, :],
                            preferred_element_type=jnp.float32)
    sc_ref[...] = acc


def _tc_sconn(nf, na, wsc):
    bn = 1000
    grid = _N // bn
    return pl.pallas_call(
        _sconn_body,
        grid=(grid,),
        in_specs=[
            pl.BlockSpec((bn, _D), lambda i: (i, 0)),
            pl.BlockSpec((bn, _DA), lambda i: (i, 0)),
            pl.BlockSpec((_D, _DA, _D), lambda i: (0, 0, 0)),
        ],
        out_specs=pl.BlockSpec((bn, _D), lambda i: (i, 0)),
        out_shape=jax.ShapeDtypeStruct((_N, _D), jnp.float32),
    )(nf, na, wsc)


def _sc_gather(nf, q, src1, dst1, half):
    mesh = plsc.VectorSubcoreMesh(core_axis_name="c", subcore_axis_name="s")

    nchh = _NCHH // _NS

    @functools.partial(
        pl.kernel,
        mesh=mesh,
        out_type=[
            jax.ShapeDtypeStruct((_HF, _D), jnp.float32),
            jax.ShapeDtypeStruct((_HF, _D), jnp.float32),
        ],
        scratch_types=[
            pltpu.VMEM((nchh * _CH,), jnp.int32),
            pltpu.VMEM((_CH, _D), jnp.float32),
            pltpu.VMEM((_CH, _D), jnp.float32),
            pltpu.VMEM_SHARED((_N, _D), jnp.float32),
            pltpu.SemaphoreType.DMA,
            pltpu.SemaphoreType.DMA,
            pltpu.SemaphoreType.DMA,
            pltpu.SemaphoreType.DMA,
        ],
    )
    def gather(nf_hbm, q_hbm, src_hbm, dst_hbm, xs_out, qd_out,
               idx_v, r0, r1, tab_sh,
               gs0, gs1, ws0, ws1):
        cid = lax.axis_index("c")
        sid = lax.axis_index("s")
        rx = (r0, r1)
        gs = (gs0, gs1)
        ws = (ws0, ws1)
        tb = pl.multiple_of(sid * 640, 8)

        def stage(tab_hbm):
            @pl.when(sid < _NS - 1)
            def _full():
                pltpu.sync_copy(tab_hbm.at[pl.ds(tb, 640)],
                                tab_sh.at[pl.ds(tb, 640)])

            @pl.when(sid == _NS - 1)
            def _last():
                pltpu.sync_copy(tab_hbm.at[pl.ds(tb, _N - 15 * 640)],
                                tab_sh.at[pl.ds(tb, _N - 15 * 640)])

        @pl.when(cid == 0)
        def _s0():
            stage(nf_hbm)

        @pl.when(cid == 1)
        def _s1():
            stage(q_hbm)

        plsc.subcore_barrier()

        def run(ih_hbm, out_hbm):
            lbase = sid * nchh * _CH
            gbase = half * _HF + lbase
            pltpu.sync_copy(ih_hbm.at[pl.ds(gbase, nchh * _CH)], idx_v)

            def fire(i, b):
                isl = idx_v.at[pl.ds(i * _CH, _CH)]
                pltpu.async_copy(tab_sh.at[isl], rx[b], gs[b])

            def handle(i, b):
                off = pl.multiple_of(lbase + i * _CH, 128)
                pltpu.make_async_copy(nf_hbm.at[pl.ds(0, _CH)],
                                      rx[b], gs[b]).wait()
                pltpu.async_copy(rx[b], out_hbm.at[pl.ds(off, _CH)], ws[b])
                pltpu.make_async_copy(rx[b], out_hbm.at[pl.ds(off, _CH)],
                                      ws[b]).wait()

            fire(0, 0)
            fire(1, 1)

            def body(g, _):
                for b in (0, 1):
                    i = 2 * g + b
                    handle(i, b)

                    @pl.when(i + 2 < nchh)
                    def _next():
                        fire(i + 2, b)
                return _

            lax.fori_loop(0, nchh // 2, body, None)

        @pl.when(cid == 0)
        def _g0():
            run(src_hbm, xs_out)

        @pl.when(cid == 1)
        def _g1():
            run(dst_hbm, qd_out)

    return gather(nf, q, src1, dst1)


def _edge_body(ee_ref, ea_ref, xs_ref, pd_ref, w1_ref, rep_ref, til_ref,
               ak_ref, av_ref, sv_ref, s2_ref, *, logc_half, ebase):
    ee = ee_ref[...]
    ea = ea_ref[...]
    h = jnp.dot(ee, w1_ref[...], preferred_element_type=jnp.float32)
    h = jnp.logaddexp(h, 0.0) - _LOG2
    hexp = jnp.dot(h, rep_ref[...], preferred_element_type=jnp.float32)
    eat = jnp.dot(ea, til_ref[...], preferred_element_type=jnp.float32)
    g = hexp * eat
    tk = jnp.dot(g, ak_ref[...], preferred_element_type=jnp.float32)
    tv = jnp.dot(g, av_ref[...], preferred_element_type=jnp.float32)
    xs = xs_ref[...]
    dot = jnp.sum(xs * tk * pd_ref[...], axis=1, keepdims=True)
    mid = xs * tv
    b = dot.shape[0]
    eidx = (ebase + pl.program_id(0) * b
            + jax.lax.broadcasted_iota(jnp.int32, (b, 1), 0))
    live = eidx < _E
    s = jnp.where(live, jnp.exp(0.5 * dot + logc_half), 0.0)
    sv_ref[...] = s * mid
    s2_ref[...] = (s * s).T


def _tc_edge(ee, ea, xs, pd, w1, rep, til, ak, av, half):
    be = 2048
    grid = _HF // be
    body = functools.partial(_edge_body, logc_half=_cutoff_log_half(),
                             ebase=half * _HF)
    hb = half * (_HF // be)
    return pl.pallas_call(
        body,
        grid=(grid,),
        in_specs=[
            pl.BlockSpec((be, _DEE), lambda i: (i + hb, 0)),
            pl.BlockSpec((be, _DEA), lambda i: (i + hb, 0)),
            pl.BlockSpec((be, _D), lambda i: (i, 0)),
            pl.BlockSpec((be, _D), lambda i: (i, 0)),
            pl.BlockSpec((_DEE, 2 * _H), lambda i: (0, 0)),
            pl.BlockSpec((2 * _H, 64), lambda i: (0, 0)),
            pl.BlockSpec((_DEA, 64), lambda i: (0, 0)),
            pl.BlockSpec((64, _D), lambda i: (0, 0)),
            pl.BlockSpec((64, _D), lambda i: (0, 0)),
        ],
        out_specs=[
            pl.BlockSpec((be, _D), lambda i: (i, 0)),
            pl.BlockSpec((1, be), lambda i: (0, i)),
        ],
        out_shape=[
            jax.ShapeDtypeStruct((_HF, _D), jnp.float32),
            jax.ShapeDtypeStruct((1, _HF), jnp.float32),
        ],
    )(ee, ea, xs, pd, w1, rep, til, ak, av)


def _sc_scatter(sv, s2, dst, zrow, zrow1, half):
    mesh = plsc.VectorSubcoreMesh(core_axis_name="c", subcore_axis_name="s")

    @functools.partial(
        pl.kernel,
        mesh=mesh,
        out_type=[
            jax.ShapeDtypeStruct((_NP, _D), jnp.float32),
            jax.ShapeDtypeStruct((_NP,), jnp.float32),
        ],
        scratch_types=[
            pltpu.VMEM((_NCH, _CH), jnp.int32),
            pltpu.VMEM((_CH, _D), jnp.float32),
            pltpu.VMEM((_CH, _D), jnp.float32),
            pltpu.VMEM((_CH,), jnp.float32),
            pltpu.VMEM((_CH,), jnp.float32),
            pltpu.VMEM_SHARED((_NP, _D), jnp.float32),
            pltpu.VMEM_SHARED((_NP,), jnp.float32),
            pltpu.SemaphoreType.DMA,
            pltpu.SemaphoreType.DMA,
            pltpu.SemaphoreType.DMA,
            pltpu.SemaphoreType.DMA,
        ],
    )
    def scatter(sv_hbm, s2_hbm, dst_hbm, zr_hbm, zr1_hbm, num_out, z_out,
                idx_v, rv0, rv1, s2a, s2c, num_sh, z_sh,
                ls0, ls1, ss0, ss1):
        cid = lax.axis_index("c")
        sid = lax.axis_index("s")
        lbase = sid * _EPW
        cbase = (half * _NS + sid) * _NCH
        rbase = pl.multiple_of(sid * _STRIPE, 8)
        rv = (rv0, rv1)
        s2b = (s2a, s2c)
        ls = (ls0, ls1)
        ss = (ss0, ss1)

        @pl.when(cid == half)
        def _active():
            pltpu.sync_copy(zr_hbm, num_sh.at[pl.ds(rbase, _STRIPE)])
            pltpu.sync_copy(zr1_hbm.at[pl.ds(rbase, _STRIPE)],
                            z_sh.at[pl.ds(rbase, _STRIPE)])
            pltpu.sync_copy(dst_hbm.at[pl.ds(cbase, _NCH)], idx_v)
            plsc.subcore_barrier()

            def fire_load(i, b):
                off = pl.multiple_of(lbase + i * _CH, 128)
                pltpu.async_copy(sv_hbm.at[pl.ds(off, _CH)], rv[b], ls[b])
                pltpu.async_copy(s2_hbm.at[0, pl.ds(off, _CH)],
                                 s2b[b], ls[b])

            fire_load(0, 0)
            fire_load(1, 1)

            def body(g, _):
                for b in (0, 1):
                    i = 2 * g + b
                    off = pl.multiple_of(lbase + i * _CH, 128)
                    pltpu.make_async_copy(sv_hbm.at[pl.ds(off, _CH)],
                                          rv[b], ls[b]).wait()
                    pltpu.make_async_copy(s2_hbm.at[0, pl.ds(off, _CH)],
                                          s2b[b], ls[b]).wait()
                    pltpu.async_copy(rv[b], num_sh.at[idx_v.at[i]], ss[b],
                                     add=True)
                    pltpu.async_copy(s2b[b], z_sh.at[idx_v.at[i]], ss[b],
                                     add=True)
                    pltpu.make_async_copy(rv[b], num_sh.at[idx_v.at[i]],
                                          ss[b]).wait()
                    pltpu.make_async_copy(s2b[b], z_sh.at[idx_v.at[i]],
                                          ss[b]).wait()

                    @pl.when(i + 2 < _NCH)
                    def _next():
                        fire_load(i + 2, b)
                return _

            lax.fori_loop(0, _NCH // 2, body, None)
            plsc.subcore_barrier()

            pltpu.sync_copy(num_sh.at[pl.ds(rbase, _STRIPE)],
                            num_out.at[pl.ds(rbase, _STRIPE)])
            pltpu.sync_copy(z_sh.at[pl.ds(rbase, _STRIPE)],
                            z_out.at[pl.ds(rbase, _STRIPE)])

    return scatter(sv, s2, dst, zrow, zrow1)


def _post_body(n0_ref, n1_ref, z0_ref, z1_ref, sc_ref, wv_ref, out_ref):
    num = n0_ref[...] + n1_ref[...]
    nv = jnp.dot(num, wv_ref[...], preferred_element_type=jnp.float32)
    z = z0_ref[...] + z1_ref[...]
    z = jnp.where(z == 0.0, 1.0, z)
    out_ref[...] = nv / jnp.sqrt(z) + sc_ref[...]


def _tc_post(n0, n1, z0, z1, sc, wv):
    bn = 1000
    grid = _N // bn
    return pl.pallas_call(
        _post_body,
        grid=(grid,),
        in_specs=[
            pl.BlockSpec((bn, _D), lambda i: (i, 0)),
            pl.BlockSpec((bn, _D), lambda i: (i, 0)),
            pl.BlockSpec((bn, 1), lambda i: (i, 0)),
            pl.BlockSpec((bn, 1), lambda i: (i, 0)),
            pl.BlockSpec((bn, _D), lambda i: (i, 0)),
            pl.BlockSpec((_D, _D), lambda i: (0, 0)),
        ],
        out_specs=pl.BlockSpec((bn, _D), lambda i: (i, 0)),
        out_shape=jax.ShapeDtypeStruct((_N, _D), jnp.float32),
    )(n0, n1, z0, z1, sc, wv)


def kernel(node_features, node_attrs, edge_embedding, edge_attrs, positions,
           edge_index, Wq, W1k, W2k, W1v, W2v, Wk, Wv, Wdot, Wsc):
    del positions
    f32 = jnp.float32
    src = edge_index[0].astype(jnp.int32)
    dst = edge_index[1].astype(jnp.int32)

    w1 = jnp.concatenate([W1k, W1v], axis=1).astype(f32)
    rep = np.zeros((2 * _H, 64), np.float32)
    til = np.zeros((_DEA, 64), np.float32)
    for hh in range(_H):
        for vv in range(_DEA):
            rep[hh, hh * _DEA + vv] = 1.0
            rep[_H + hh, 32 + hh * _DEA + vv] = 1.0
            til[vv, hh * _DEA + vv] = 1.0
            til[vv, 32 + hh * _DEA + vv] = 1.0
    rep = jnp.asarray(rep)
    til = jnp.asarray(til)
    akm = W2k.reshape(_H, _D, _DEA).transpose(0, 2, 1).reshape(32, _D)
    avm = W2v.reshape(_H, _D, _DEA).transpose(0, 2, 1).reshape(32, _D)
    scale = 1.0 / np.sqrt(_DEA)
    ak = jnp.zeros((64, _D), f32).at[:32].set(akm * scale)
    av = jnp.zeros((64, _D), f32).at[32:].set(avm * scale)

    zrow = jnp.zeros((_STRIPE, _D), f32)
    zrow1 = jnp.zeros((_NP,), f32)

    padn = _EP - _E
    src1 = jnp.pad(src, (0, padn))
    dst1 = jnp.pad(dst, (0, padn))
    dst2d = dst1.reshape(_EP // _CH, _CH)
    eep = jnp.pad(edge_embedding, ((0, padn), (0, 0)))
    eap = jnp.pad(edge_attrs, ((0, padn), (0, 0)))

    wq128 = (Wq @ Wdot @ Wk.T).astype(f32)
    p = _tc_ptab(node_features, wq128)
    xs0, pd0 = _sc_gather(node_features, p, src1, dst1, 0)
    xs1, pd1 = _sc_gather(node_features, p, src1, dst1, 1)
    sc = _tc_sconn(node_features, node_attrs, Wsc)
    sv0, s20 = _tc_edge(eep, eap, xs0, pd0, w1, rep, til, ak, av, 0)
    sv1, s21 = _tc_edge(eep, eap, xs1, pd1, w1, rep, til, ak, av, 1)
    n0, za = _sc_scatter(sv0, s20, dst2d, zrow, zrow1, 0)
    n1, zb = _sc_scatter(sv1, s21, dst2d, zrow, zrow1, 1)
    z0 = za[:_N, None]
    z1 = zb[:_N, None]
    return _tc_post(n0, n1, z0, z1, sc, Wv)


# speedup vs baseline: 1.7095x; 1.1024x over previous
import functools

import jax
import jax.numpy as jnp
import numpy as np
from jax import lax
from jax.experimental import pallas as pl
from jax.experimental.pallas import tpu as pltpu
from jax.experimental.pallas import tpu_sc as plsc

_N = 10000
_E = 160000
_D = 128
_DA = 16
_DEA = 4
_DEE = 16
_DQK = 32
_H = 8

_NC = 2
_NS = 16
_NW = _NC * _NS
_EP = 163840
_EPW = _EP // _NW
_CH = 128
_NCH = _EPW // _CH
_NP = 10240
_STRIPE = _NP // _NS
_HF = _EP // 2
_NCHH = _HF // _CH
_CPW = _NCHH // _NW

_LOG2 = float(np.log(2.0))


def _cutoff_log_half() -> float:
    el = np.sqrt(np.float32(1e-12))
    xc = np.float32(10.0) * (np.float32(1.0) - el / np.float32(5.0))
    return float(-0.5 / float(xc))


def _ptab_body(nf_ref, wq_ref, q_ref):
    q_ref[...] = jnp.dot(nf_ref[...], wq_ref[...],
                         preferred_element_type=jnp.float32)


def _tc_ptab(nf, wq):
    bn = 1000
    grid = _N // bn
    return pl.pallas_call(
        _ptab_body,
        grid=(grid,),
        in_specs=[
            pl.BlockSpec((bn, _D), lambda i: (i, 0)),
            pl.BlockSpec((_D, _D), lambda i: (0, 0)),
        ],
        out_specs=pl.BlockSpec((bn, _D), lambda i: (i, 0)),
        out_shape=jax.ShapeDtypeStruct((_N, _D), jnp.float32),
    )(nf, wq)


def _sconn_body(nf_ref, na_ref, wsc_ref, sc_ref):
    nf = nf_ref[...]
    na = na_ref[...]
    acc = jnp.zeros((nf.shape[0], _D), jnp.float32)
    for b in range(_DA):
        acc = acc + jnp.dot(na[:, b:b + 1] * nf, wsc_ref[:, b, :],
                            preferred_element_type=jnp.float32)
    sc_ref[...] = acc


def _tc_sconn(nf, na, wsc):
    bn = 1000
    grid = _N // bn
    return pl.pallas_call(
        _sconn_body,
        grid=(grid,),
        in_specs=[
            pl.BlockSpec((bn, _D), lambda i: (i, 0)),
            pl.BlockSpec((bn, _DA), lambda i: (i, 0)),
            pl.BlockSpec((_D, _DA, _D), lambda i: (0, 0, 0)),
        ],
        out_specs=pl.BlockSpec((bn, _D), lambda i: (i, 0)),
        out_shape=jax.ShapeDtypeStruct((_N, _D), jnp.float32),
    )(nf, na, wsc)


def _sc_gather(nf, q, src1, dst1, half):
    mesh = plsc.VectorSubcoreMesh(core_axis_name="c", subcore_axis_name="s")

    nchh = _NCHH // _NS

    @functools.partial(
        pl.kernel,
        mesh=mesh,
        out_type=[
            jax.ShapeDtypeStruct((_HF, _D), jnp.float32),
            jax.ShapeDtypeStruct((_HF, _D), jnp.float32),
        ],
        scratch_types=[
            pltpu.VMEM((nchh * _CH,), jnp.int32),
            pltpu.VMEM((_CH, _D), jnp.float32),
            pltpu.VMEM((_CH, _D), jnp.float32),
            pltpu.VMEM_SHARED((_N, _D), jnp.float32),
            pltpu.SemaphoreType.DMA,
            pltpu.SemaphoreType.DMA,
            pltpu.SemaphoreType.DMA,
            pltpu.SemaphoreType.DMA,
        ],
    )
    def gather(nf_hbm, q_hbm, src_hbm, dst_hbm, xs_out, qd_out,
               idx_v, r0, r1, tab_sh,
               gs0, gs1, ws0, ws1):
        cid = lax.axis_index("c")
        sid = lax.axis_index("s")
        rx = (r0, r1)
        gs = (gs0, gs1)
        ws = (ws0, ws1)
        tb = pl.multiple_of(sid * 640, 8)

        def stage(tab_hbm):
            @pl.when(sid < _NS - 1)
            def _full():
                pltpu.sync_copy(tab_hbm.at[pl.ds(tb, 640)],
                                tab_sh.at[pl.ds(tb, 640)])

            @pl.when(sid == _NS - 1)
            def _last():
                pltpu.sync_copy(tab_hbm.at[pl.ds(tb, _N - 15 * 640)],
                                tab_sh.at[pl.ds(tb, _N - 15 * 640)])

        @pl.when(cid == 0)
        def _s0():
            stage(nf_hbm)

        @pl.when(cid == 1)
        def _s1():
            stage(q_hbm)

        plsc.subcore_barrier()

        def run(ih_hbm, out_hbm):
            lbase = sid * nchh * _CH
            gbase = half * _HF + lbase
            pltpu.sync_copy(ih_hbm.at[pl.ds(gbase, nchh * _CH)], idx_v)

            def fire(i, b):
                isl = idx_v.at[pl.ds(i * _CH, _CH)]
                pltpu.async_copy(tab_sh.at[isl], rx[b], gs[b])

            def handle(i, b):
                off = pl.multiple_of(lbase + i * _CH, 128)
                pltpu.make_async_copy(nf_hbm.at[pl.ds(0, _CH)],
                                      rx[b], gs[b]).wait()
                pltpu.async_copy(rx[b], out_hbm.at[pl.ds(off, _CH)], ws[b])
                pltpu.make_async_copy(rx[b], out_hbm.at[pl.ds(off, _CH)],
                                      ws[b]).wait()

            fire(0, 0)
            fire(1, 1)

            def body(g, _):
                for b in (0, 1):
                    i = 2 * g + b
                    handle(i, b)

                    @pl.when(i + 2 < nchh)
                    def _next():
                        fire(i + 2, b)
                return _

            lax.fori_loop(0, nchh // 2, body, None)

        @pl.when(cid == 0)
        def _g0():
            run(src_hbm, xs_out)

        @pl.when(cid == 1)
        def _g1():
            run(dst_hbm, qd_out)

    return gather(nf, q, src1, dst1)


def _edge_body(ee_ref, ea_ref, xs_ref, pd_ref, w1_ref, rep_ref, til_ref,
               ak_ref, av_ref, sv_ref, s2_ref, *, logc_half, ebase):
    ee = ee_ref[...]
    ea = ea_ref[...]
    h = jnp.dot(ee, w1_ref[...], preferred_element_type=jnp.float32)
    h = jnp.logaddexp(h, 0.0) - _LOG2
    hexp = jnp.dot(h, rep_ref[...], preferred_element_type=jnp.float32)
    eat = jnp.dot(ea, til_ref[...], preferred_element_type=jnp.float32)
    g = hexp * eat
    tk = jnp.dot(g, ak_ref[...], preferred_element_type=jnp.float32)
    tv = jnp.dot(g, av_ref[...], preferred_element_type=jnp.float32)
    xs = xs_ref[...]
    dot = jnp.sum(xs * tk * pd_ref[...], axis=1, keepdims=True)
    mid = xs * tv
    b = dot.shape[0]
    eidx = (ebase + pl.program_id(0) * b
            + jax.lax.broadcasted_iota(jnp.int32, (b, 1), 0))
    live = eidx < _E
    s = jnp.where(live, jnp.exp(0.5 * dot + logc_half), 0.0)
    sv_ref[...] = s * mid
    s2_ref[...] = (s * s).T


def _tc_edge(ee, ea, xs, pd, w1, rep, til, ak, av, half):
    be = 1280
    grid = _HF // be
    body = functools.partial(_edge_body, logc_half=_cutoff_log_half(),
                             ebase=half * _HF)
    hb = half * (_HF // be)
    lastlive = _E // be - 1
    return pl.pallas_call(
        body,
        grid=(grid,),
        in_specs=[
            pl.BlockSpec((be, _DEE),
                         lambda i: (jnp.minimum(i + hb, lastlive), 0)),
            pl.BlockSpec((be, _DEA),
                         lambda i: (jnp.minimum(i + hb, lastlive), 0)),
            pl.BlockSpec((be, _D), lambda i: (i, 0)),
            pl.BlockSpec((be, _D), lambda i: (i, 0)),
            pl.BlockSpec((_DEE, 2 * _H), lambda i: (0, 0)),
            pl.BlockSpec((2 * _H, 64), lambda i: (0, 0)),
            pl.BlockSpec((_DEA, 64), lambda i: (0, 0)),
            pl.BlockSpec((64, _D), lambda i: (0, 0)),
            pl.BlockSpec((64, _D), lambda i: (0, 0)),
        ],
        out_specs=[
            pl.BlockSpec((be, _D), lambda i: (i, 0)),
            pl.BlockSpec((1, be), lambda i: (0, i)),
        ],
        out_shape=[
            jax.ShapeDtypeStruct((_HF, _D), jnp.float32),
            jax.ShapeDtypeStruct((1, _HF), jnp.float32),
        ],
    )(ee, ea, xs, pd, w1, rep, til, ak, av)


def _sc_scatter(sv, s2, dst, zrow, zrow1, half):
    mesh = plsc.VectorSubcoreMesh(core_axis_name="c", subcore_axis_name="s")

    @functools.partial(
        pl.kernel,
        mesh=mesh,
        out_type=[
            jax.ShapeDtypeStruct((_NP, _D), jnp.float32),
            jax.ShapeDtypeStruct((_NP,), jnp.float32),
        ],
        scratch_types=[
            pltpu.VMEM((_NCH, _CH), jnp.int32),
            pltpu.VMEM((_CH, _D), jnp.float32),
            pltpu.VMEM((_CH, _D), jnp.float32),
            pltpu.VMEM((_CH,), jnp.float32),
            pltpu.VMEM((_CH,), jnp.float32),
            pltpu.VMEM_SHARED((_NP, _D), jnp.float32),
            pltpu.VMEM_SHARED((_NP,), jnp.float32),
            pltpu.SemaphoreType.DMA,
            pltpu.SemaphoreType.DMA,
            pltpu.SemaphoreType.DMA,
            pltpu.SemaphoreType.DMA,
        ],
    )
    def scatter(sv_hbm, s2_hbm, dst_hbm, zr_hbm, zr1_hbm, num_out, z_out,
                idx_v, rv0, rv1, s2a, s2c, num_sh, z_sh,
                ls0, ls1, ss0, ss1):
        cid = lax.axis_index("c")
        sid = lax.axis_index("s")
        lbase = sid * _EPW
        cbase = (half * _NS + sid) * _NCH
        rbase = pl.multiple_of(sid * _STRIPE, 8)
        rv = (rv0, rv1)
        s2b = (s2a, s2c)
        ls = (ls0, ls1)
        ss = (ss0, ss1)

        @pl.when(cid == half)
        def _active():
            pltpu.sync_copy(zr_hbm, num_sh.at[pl.ds(rbase, _STRIPE)])
            pltpu.sync_copy(zr1_hbm.at[pl.ds(rbase, _STRIPE)],
                            z_sh.at[pl.ds(rbase, _STRIPE)])
            pltpu.sync_copy(dst_hbm.at[pl.ds(cbase, _NCH)], idx_v)
            plsc.subcore_barrier()

            def fire_load(i, b):
                off = pl.multiple_of(lbase + i * _CH, 128)
                pltpu.async_copy(sv_hbm.at[pl.ds(off, _CH)], rv[b], ls[b])
                pltpu.async_copy(s2_hbm.at[0, pl.ds(off, _CH)],
                                 s2b[b], ls[b])

            fire_load(0, 0)
            fire_load(1, 1)

            def body(g, _):
                for b in (0, 1):
                    i = 2 * g + b
                    off = pl.multiple_of(lbase + i * _CH, 128)
                    pltpu.make_async_copy(sv_hbm.at[pl.ds(off, _CH)],
                                          rv[b], ls[b]).wait()
                    pltpu.make_async_copy(s2_hbm.at[0, pl.ds(off, _CH)],
                                          s2b[b], ls[b]).wait()
                    pltpu.async_copy(rv[b], num_sh.at[idx_v.at[i]], ss[b],
                                     add=True)
                    pltpu.async_copy(s2b[b], z_sh.at[idx_v.at[i]], ss[b],
                                     add=True)
                    pltpu.make_async_copy(rv[b], num_sh.at[idx_v.at[i]],
                                          ss[b]).wait()
                    pltpu.make_async_copy(s2b[b], z_sh.at[idx_v.at[i]],
                                          ss[b]).wait()

                    @pl.when(i + 2 < _NCH)
                    def _next():
                        fire_load(i + 2, b)
                return _

            lax.fori_loop(0, _NCH // 2, body, None)
            plsc.subcore_barrier()

            pltpu.sync_copy(num_sh.at[pl.ds(rbase, _STRIPE)],
                            num_out.at[pl.ds(rbase, _STRIPE)])
            pltpu.sync_copy(z_sh.at[pl.ds(rbase, _STRIPE)],
                            z_out.at[pl.ds(rbase, _STRIPE)])

    return scatter(sv, s2, dst, zrow, zrow1)


def _post_body(n0_ref, n1_ref, z0_ref, z1_ref, sc_ref, wv_ref, out_ref):
    num = n0_ref[...] + n1_ref[...]
    nv = jnp.dot(num, wv_ref[...], preferred_element_type=jnp.float32)
    z = z0_ref[...] + z1_ref[...]
    z = jnp.where(z == 0.0, 1.0, z)
    out_ref[...] = nv / jnp.sqrt(z) + sc_ref[...]


def _tc_post(n0, n1, z0, z1, sc, wv):
    bn = 1000
    grid = _N // bn
    return pl.pallas_call(
        _post_body,
        grid=(grid,),
        in_specs=[
            pl.BlockSpec((bn, _D), lambda i: (i, 0)),
            pl.BlockSpec((bn, _D), lambda i: (i, 0)),
            pl.BlockSpec((bn, 1), lambda i: (i, 0)),
            pl.BlockSpec((bn, 1), lambda i: (i, 0)),
            pl.BlockSpec((bn, _D), lambda i: (i, 0)),
            pl.BlockSpec((_D, _D), lambda i: (0, 0)),
        ],
        out_specs=pl.BlockSpec((bn, _D), lambda i: (i, 0)),
        out_shape=jax.ShapeDtypeStruct((_N, _D), jnp.float32),
    )(n0, n1, z0, z1, sc, wv)


def kernel(node_features, node_attrs, edge_embedding, edge_attrs, positions,
           edge_index, Wq, W1k, W2k, W1v, W2v, Wk, Wv, Wdot, Wsc):
    del positions
    f32 = jnp.float32
    src = edge_index[0].astype(jnp.int32)
    dst = edge_index[1].astype(jnp.int32)

    w1 = jnp.concatenate([W1k, W1v], axis=1).astype(f32)
    rep = np.zeros((2 * _H, 64), np.float32)
    til = np.zeros((_DEA, 64), np.float32)
    for hh in range(_H):
        for vv in range(_DEA):
            rep[hh, hh * _DEA + vv] = 1.0
            rep[_H + hh, 32 + hh * _DEA + vv] = 1.0
            til[vv, hh * _DEA + vv] = 1.0
            til[vv, 32 + hh * _DEA + vv] = 1.0
    rep = jnp.asarray(rep)
    til = jnp.asarray(til)
    akm = W2k.reshape(_H, _D, _DEA).transpose(0, 2, 1).reshape(32, _D)
    avm = W2v.reshape(_H, _D, _DEA).transpose(0, 2, 1).reshape(32, _D)
    scale = 1.0 / np.sqrt(_DEA)
    ak = jnp.zeros((64, _D), f32).at[:32].set(akm * scale)
    av = jnp.zeros((64, _D), f32).at[32:].set(avm * scale)

    zrow = jnp.zeros((_STRIPE, _D), f32)
    zrow1 = jnp.zeros((_NP,), f32)

    padn = _EP - _E
    src1 = jnp.pad(src, (0, padn))
    dst1 = jnp.pad(dst, (0, padn))
    dst2d = dst1.reshape(_EP // _CH, _CH)

    wq128 = (Wq @ Wdot @ Wk.T).astype(f32)
    p = _tc_ptab(node_features, wq128)
    xs0, pd0 = _sc_gather(node_features, p, src1, dst1, 0)
    xs1, pd1 = _sc_gather(node_features, p, src1, dst1, 1)
    sc = _tc_sconn(node_features, node_attrs, Wsc)
    sv0, s20 = _tc_edge(edge_embedding, edge_attrs, xs0, pd0,
                        w1, rep, til, ak, av, 0)
    sv1, s21 = _tc_edge(edge_embedding, edge_attrs, xs1, pd1,
                        w1, rep, til, ak, av, 1)
    n0, za = _sc_scatter(sv0, s20, dst2d, zrow, zrow1, 0)
    n1, zb = _sc_scatter(sv1, s21, dst2d, zrow, zrow1, 1)
    z0 = za[:_N, None]
    z1 = zb[:_N, None]
    return _tc_post(n0, n1, z0, z1, sc, Wv)
